# baseline (speedup 1.0000x reference)
import functools

import jax
import jax.numpy as jnp
from jax import lax
from jax.experimental import pallas as pl
from jax.experimental.pallas import tpu as pltpu
from jax.experimental.pallas import tpu_sc as plsc

N_NODES = 10000
N_PAD = 10240
D_HID = 16
E_EDGES = 320000
NW = 32
CHUNK = 512
EPT = 10240
NCHUNK = EPT // CHUNK
ROWS_PT = N_PAD // 16
NBUF = 4

_SC_MESH = plsc.VectorSubcoreMesh(core_axis_name="c", subcore_axis_name="s")
_SC_PARAMS = pltpu.CompilerParams(use_tc_tiling_on_sc=False)


@functools.partial(
    pl.kernel,
    out_type=jax.ShapeDtypeStruct((2, N_PAD), jnp.float32),
    mesh=_SC_MESH,
    scratch_types=[
        pltpu.VMEM((NCHUNK, CHUNK), jnp.int32),
        pltpu.VMEM((CHUNK,), jnp.float32),
        [pltpu.SemaphoreType.DMA for _ in range(NBUF)],
        pltpu.VMEM_SHARED((N_PAD,), jnp.float32),
    ],
    compiler_params=_SC_PARAMS,
)
def _deg_kernel(dst2d, ones_init, zeros_init, out, didx, ones_v, ssems, dacc):
    cid = lax.axis_index("c")
    sid = lax.axis_index("s")
    w = cid * 16 + sid
    pltpu.sync_copy(dst2d.at[pl.ds(w * NCHUNK, NCHUNK)], didx)
    pltpu.sync_copy(ones_init.at[pl.ds(0, CHUNK)], ones_v)
    row0 = sid * ROWS_PT

    @pl.when(cid == 0)
    def _():
        pltpu.sync_copy(ones_init.at[pl.ds(row0, ROWS_PT)],
                        dacc.at[pl.ds(row0, ROWS_PT)])

    @pl.when(cid == 1)
    def _():
        pltpu.sync_copy(zeros_init.at[pl.ds(row0, ROWS_PT)],
                        dacc.at[pl.ds(row0, ROWS_PT)])

    plsc.subcore_barrier()

    def body(i, carry):
        jj = i * NBUF
        sds = [
            pltpu.async_copy(ones_v, dacc.at[didx.at[jj + b]], ssems[b],
                             add=True)
            for b in range(NBUF)
        ]
        for d in sds:
            d.wait()
        return carry

    lax.fori_loop(0, NCHUNK // NBUF, body, 0)
    plsc.subcore_barrier()
    pltpu.sync_copy(dacc.at[pl.ds(row0, ROWS_PT)],
                    out.at[cid, pl.ds(row0, ROWS_PT)])


@functools.partial(
    pl.kernel,
    out_type=jax.ShapeDtypeStruct((2, N_PAD, D_HID), jnp.float32),
    mesh=_SC_MESH,
    scratch_types=[
        pltpu.VMEM((NCHUNK, CHUNK), jnp.int32),
        pltpu.VMEM((NCHUNK, CHUNK), jnp.int32),
        [pltpu.VMEM((CHUNK, D_HID), jnp.float32) for _ in range(NBUF)],
        [pltpu.SemaphoreType.DMA for _ in range(NBUF)],
        [pltpu.SemaphoreType.DMA for _ in range(NBUF)],
        pltpu.VMEM_SHARED((N_PAD, D_HID), jnp.float32),
        pltpu.VMEM_SHARED((N_PAD, D_HID), jnp.float32),
    ],
    compiler_params=_SC_PARAMS,
)
def _agg_kernel(src2d, dst2d, hp, zinit, out, sidx, didx, bufs, gsems, ssems,
                acc, tbl):
    cid = lax.axis_index("c")
    sid = lax.axis_index("s")
    w = cid * 16 + sid
    pltpu.sync_copy(src2d.at[pl.ds(w * NCHUNK, NCHUNK)], sidx)
    pltpu.sync_copy(dst2d.at[pl.ds(w * NCHUNK, NCHUNK)], didx)
    row0 = sid * ROWS_PT

    pltpu.sync_copy(hp.at[pl.ds(row0, ROWS_PT)], tbl.at[pl.ds(row0, ROWS_PT)])
    pltpu.sync_copy(zinit.at[pl.ds(row0, ROWS_PT)],
                    acc.at[pl.ds(row0, ROWS_PT)])

    plsc.subcore_barrier()

    def body(i, carry):
        jj = i * NBUF
        gds = [
            pltpu.async_copy(tbl.at[sidx.at[jj + b]], bufs[b], gsems[b])
            for b in range(NBUF)
        ]
        sds = []
        for b in range(NBUF):
            gds[b].wait()
            sds.append(
                pltpu.async_copy(bufs[b], acc.at[didx.at[jj + b]], ssems[b],
                                 add=True))
        for d in sds:
            d.wait()
        return carry

    lax.fori_loop(0, NCHUNK // NBUF, body, 0)
    plsc.subcore_barrier()
    pltpu.sync_copy(acc.at[pl.ds(row0, ROWS_PT)],
                    out.at[cid, pl.ds(row0, ROWS_PT)])


def _mm_body(x_ref, w_ref, o_ref):
    o_ref[...] = jnp.dot(x_ref[...], w_ref[...],
                         preferred_element_type=jnp.float32)


def _scale_body(h0_ref, d0_ref, d1_ref, dis_ref, hp_ref):
    dis = lax.rsqrt(d0_ref[...] + d1_ref[...])
    dis_ref[...] = dis
    hp_ref[...] = h0_ref[...] * dis


def _relu_scale_body(p_ref, hp_ref, dis_ref, g1_ref):
    dis = dis_ref[...]
    g1_ref[...] = dis * jnp.maximum(
        dis * (p_ref[0] + p_ref[1] + hp_ref[...]), 0.0)


def _final_body(q_ref, g1_ref, dis_ref, w1_ref, o_ref):
    q = dis_ref[...] * (q_ref[0] + q_ref[1] + g1_ref[...])
    o_ref[...] = jnp.dot(q, w1_ref[...], preferred_element_type=jnp.float32)


def kernel(x, edge_index, W0, W1):
    f32 = jnp.float32
    src = edge_index[0]
    dst = edge_index[1]

    pad = NW * EPT - E_EDGES
    pad_src = jnp.zeros((pad,), jnp.int32)
    pad_dst = (N_NODES + (jnp.arange(pad, dtype=jnp.int32) % 16))
    src2d = jnp.concatenate([src, pad_src]).reshape(NW * NCHUNK, CHUNK)
    dst2d = jnp.concatenate([dst, pad_dst]).reshape(NW * NCHUNK, CHUNK)
    x_pad = jnp.pad(x, ((0, N_PAD - N_NODES), (0, 0)))
    ones1 = jnp.ones((N_PAD,), f32)
    zeros1 = jnp.zeros((N_PAD,), f32)
    zeros16 = jnp.zeros((N_PAD, D_HID), f32)

    h0 = pl.pallas_call(
        _mm_body,
        grid=(8,),
        in_specs=[
            pl.BlockSpec((N_PAD // 8, 128), lambda i: (i, 0)),
            pl.BlockSpec((128, D_HID), lambda i: (0, 0)),
        ],
        out_specs=pl.BlockSpec((N_PAD // 8, D_HID), lambda i: (i, 0)),
        out_shape=jax.ShapeDtypeStruct((N_PAD, D_HID), f32),
    )(x_pad, W0)

    deg = _deg_kernel(dst2d, ones1, zeros1)

    d0 = deg[0].reshape(N_PAD, 1)
    d1 = deg[1].reshape(N_PAD, 1)
    dis, hp = pl.pallas_call(
        _scale_body,
        out_shape=(
            jax.ShapeDtypeStruct((N_PAD, 1), f32),
            jax.ShapeDtypeStruct((N_PAD, D_HID), f32),
        ),
    )(h0, d0, d1)

    p = _agg_kernel(src2d, dst2d, hp, zeros16)

    g1 = pl.pallas_call(
        _relu_scale_body,
        out_shape=jax.ShapeDtypeStruct((N_PAD, D_HID), f32),
    )(p, hp, dis)

    q = _agg_kernel(src2d, dst2d, g1, zeros16)

    out = pl.pallas_call(
        _final_body,
        out_shape=jax.ShapeDtypeStruct((N_PAD, 7), f32),
    )(q, g1, dis, W1)

    return out[:N_NODES]


# speedup vs baseline: 1.1052x; 1.1052x over previous
import functools

import jax
import jax.numpy as jnp
from jax import lax
from jax.experimental import pallas as pl
from jax.experimental.pallas import tpu as pltpu
from jax.experimental.pallas import tpu_sc as plsc

N_NODES = 10000
N_PAD = 10240
D_HID = 16
E_EDGES = 320000
CHUNK = 128
NROWS2D = E_EDGES // CHUNK
BASECH = NROWS2D // 32
NBUF = 6
ROWS_FULL = 640
ROWS_LAST = N_NODES - 15 * ROWS_FULL

_SC_MESH = plsc.VectorSubcoreMesh(core_axis_name="c", subcore_axis_name="s")
_SC_PARAMS = pltpu.CompilerParams(use_tc_tiling_on_sc=False,
                                  needs_layout_passes=False)


def _rsqrt16(d):
    i = plsc.bitcast(d, jnp.int32)
    i = jnp.full((16,), 0x5F3759DF, jnp.int32) - lax.shift_right_logical(i, 1)
    y = plsc.bitcast(i, jnp.float32)
    half = 0.5 * d
    for _ in range(3):
        y = y * (1.5 - half * y * y)
    return y


_GATHER_DNUMS = lax.GatherDimensionNumbers(
    offset_dims=(), collapsed_slice_dims=(0,), start_index_map=(0,))


def _bcast_lane(v, k):
    idx = jnp.full((16, 1), k, jnp.int32)
    return lax.gather(v, idx, _GATHER_DNUMS, slice_sizes=(1,),
                      mode=lax.GatherScatterMode.PROMISE_IN_BOUNDS)


@functools.partial(
    pl.kernel,
    out_type=jax.ShapeDtypeStruct((2, N_PAD), jnp.float32),
    mesh=_SC_MESH,
    scratch_types=[
        pltpu.VMEM((BASECH + 1, CHUNK), jnp.int32),
        pltpu.VMEM((CHUNK,), jnp.float32),
        [pltpu.SemaphoreType.DMA for _ in range(NBUF)],
        pltpu.VMEM_SHARED((N_PAD,), jnp.float32),
    ],
    compiler_params=_SC_PARAMS,
)
def _deg_kernel(edge2d, ones_init, zeros_init, out, didx, ones_v, ssems,
                dacc):
    cid = lax.axis_index("c")
    sid = lax.axis_index("s")
    w = cid * 16 + sid
    base = w * BASECH + jnp.minimum(w, 4)
    extra = w < 4
    pltpu.sync_copy(edge2d.at[1, pl.ds(base, BASECH)],
                    didx.at[pl.ds(0, BASECH)])

    @pl.when(extra)
    def _():
        pltpu.sync_copy(edge2d.at[1, pl.ds(base + BASECH, 1)],
                        didx.at[pl.ds(BASECH, 1)])

    pltpu.sync_copy(ones_init.at[pl.ds(0, CHUNK)], ones_v)
    row0 = sid * ROWS_FULL

    @pl.when(cid == 0)
    def _():
        pltpu.sync_copy(ones_init.at[pl.ds(row0, ROWS_FULL)],
                        dacc.at[pl.ds(row0, ROWS_FULL)])

    @pl.when(cid == 1)
    def _():
        pltpu.sync_copy(zeros_init.at[pl.ds(row0, ROWS_FULL)],
                        dacc.at[pl.ds(row0, ROWS_FULL)])

    plsc.subcore_barrier()

    def body(i, carry):
        jj = i * NBUF
        sds = [
            pltpu.async_copy(ones_v, dacc.at[didx.at[jj + b]], ssems[b],
                             add=True)
            for b in range(NBUF)
        ]
        for d in sds:
            d.wait()
        return carry

    lax.fori_loop(0, BASECH // NBUF, body, 0)

    @pl.when(extra)
    def _():
        pltpu.sync_copy(ones_v, dacc.at[didx.at[BASECH]], add=True)

    plsc.subcore_barrier()
    pltpu.sync_copy(dacc.at[pl.ds(row0, ROWS_FULL)],
                    out.at[cid, pl.ds(row0, ROWS_FULL)])


@functools.partial(
    pl.kernel,
    out_type=(
        jax.ShapeDtypeStruct((2, N_NODES, D_HID), jnp.float32),
        jax.ShapeDtypeStruct((N_NODES, D_HID), jnp.float32),
    ),
    mesh=_SC_MESH,
    scratch_types=[
        pltpu.VMEM((BASECH + 1, CHUNK), jnp.int32),
        pltpu.VMEM((BASECH + 1, CHUNK), jnp.int32),
        pltpu.VMEM((ROWS_FULL,), jnp.float32),
        pltpu.VMEM((ROWS_FULL,), jnp.float32),
        pltpu.VMEM((ROWS_FULL, D_HID), jnp.float32),
        pltpu.VMEM((ROWS_FULL, D_HID), jnp.float32),
        pltpu.VMEM((ROWS_FULL, D_HID), jnp.float32),
        pltpu.VMEM((ROWS_FULL, D_HID), jnp.float32),
        pltpu.VMEM((ROWS_FULL, D_HID), jnp.float32),
        [pltpu.VMEM((CHUNK, D_HID), jnp.float32) for _ in range(NBUF)],
        [pltpu.SemaphoreType.DMA for _ in range(NBUF)],
        [pltpu.SemaphoreType.DMA for _ in range(NBUF)],
        pltpu.VMEM_SHARED((N_PAD, D_HID), jnp.float32),
        pltpu.VMEM_SHARED((N_PAD, D_HID), jnp.float32),
    ],
    compiler_params=_SC_PARAMS,
)
def _agg_kernel(hin, deg, edge2d, zeros16, parts, selfterm, sidx, didx, dg0,
                dg1, hbuf, sbuf, abuf, obuf, stbuf, bufs, gsems, ssems, acc,
                tbl):
    cid = lax.axis_index("c")
    sid = lax.axis_index("s")
    w = cid * 16 + sid
    base = w * BASECH + jnp.minimum(w, 4)
    extra = w < 4
    row0 = sid * ROWS_FULL
    last = sid == 15
    nvec = jnp.where(last, ROWS_LAST // 16, ROWS_FULL // 16)

    pltpu.sync_copy(edge2d.at[0, pl.ds(base, BASECH)],
                    sidx.at[pl.ds(0, BASECH)])
    pltpu.sync_copy(edge2d.at[1, pl.ds(base, BASECH)],
                    didx.at[pl.ds(0, BASECH)])

    @pl.when(extra)
    def _():
        pltpu.sync_copy(edge2d.at[0, pl.ds(base + BASECH, 1)],
                        sidx.at[pl.ds(BASECH, 1)])
        pltpu.sync_copy(edge2d.at[1, pl.ds(base + BASECH, 1)],
                        didx.at[pl.ds(BASECH, 1)])

    @pl.when(~last)
    def _():
        pltpu.sync_copy(deg.at[0, pl.ds(row0, ROWS_FULL)], dg0)
        pltpu.sync_copy(deg.at[1, pl.ds(row0, ROWS_FULL)], dg1)
        pltpu.sync_copy(hin.at[pl.ds(row0, ROWS_FULL)], hbuf)
        pltpu.sync_copy(zeros16.at[pl.ds(row0, ROWS_FULL)],
                        acc.at[pl.ds(row0, ROWS_FULL)])

    @pl.when(last)
    def _():
        pltpu.sync_copy(deg.at[0, pl.ds(row0, ROWS_LAST)],
                        dg0.at[pl.ds(0, ROWS_LAST)])
        pltpu.sync_copy(deg.at[1, pl.ds(row0, ROWS_LAST)],
                        dg1.at[pl.ds(0, ROWS_LAST)])
        pltpu.sync_copy(hin.at[pl.ds(row0, ROWS_LAST)],
                        hbuf.at[pl.ds(0, ROWS_LAST)])
        pltpu.sync_copy(zeros16.at[pl.ds(row0, ROWS_LAST)],
                        acc.at[pl.ds(row0, ROWS_LAST)])

    def scale_body(i, carry):
        r = i * 16
        dis = _rsqrt16(dg0[pl.ds(r, 16)] + dg1[pl.ds(r, 16)])
        for k in range(16):
            sbuf[r + k] = hbuf[r + k] * _bcast_lane(dis, k)
        return carry

    lax.fori_loop(0, nvec, scale_body, 0)

    @pl.when(~last)
    def _():
        pltpu.sync_copy(sbuf, tbl.at[pl.ds(row0, ROWS_FULL)])

    @pl.when(last)
    def _():
        pltpu.sync_copy(sbuf.at[pl.ds(0, ROWS_LAST)],
                        tbl.at[pl.ds(row0, ROWS_LAST)])

    plsc.subcore_barrier()

    def body(i, carry):
        jj = i * NBUF
        gds = [
            pltpu.async_copy(tbl.at[sidx.at[jj + b]], bufs[b], gsems[b])
            for b in range(NBUF)
        ]
        sds = []
        for b in range(NBUF):
            gds[b].wait()
            sds.append(
                pltpu.async_copy(bufs[b], acc.at[didx.at[jj + b]], ssems[b],
                                 add=True))
        for d in sds:
            d.wait()
        return carry

    lax.fori_loop(0, BASECH // NBUF, body, 0)

    @pl.when(extra)
    def _():
        pltpu.sync_copy(tbl.at[sidx.at[BASECH]], bufs[0])
        pltpu.sync_copy(bufs[0], acc.at[didx.at[BASECH]], add=True)

    plsc.subcore_barrier()

    @pl.when(~last)
    def _():
        pltpu.sync_copy(acc.at[pl.ds(row0, ROWS_FULL)], abuf)

    @pl.when(last)
    def _():
        pltpu.sync_copy(acc.at[pl.ds(row0, ROWS_LAST)],
                        abuf.at[pl.ds(0, ROWS_LAST)])

    def out_body(i, carry):
        r = i * 16
        dis = _rsqrt16(dg0[pl.ds(r, 16)] + dg1[pl.ds(r, 16)])
        for k in range(16):
            d16 = _bcast_lane(dis, k)
            obuf[r + k] = abuf[r + k] * d16
            stbuf[r + k] = sbuf[r + k] * d16
        return carry

    lax.fori_loop(0, nvec, out_body, 0)

    @pl.when(~last)
    def _():
        pltpu.sync_copy(obuf, parts.at[cid, pl.ds(row0, ROWS_FULL)])

    @pl.when(last)
    def _():
        pltpu.sync_copy(obuf.at[pl.ds(0, ROWS_LAST)],
                        parts.at[cid, pl.ds(row0, ROWS_LAST)])

    @pl.when((cid == 0) & ~last)
    def _():
        pltpu.sync_copy(stbuf, selfterm.at[pl.ds(row0, ROWS_FULL)])

    @pl.when((cid == 0) & last)
    def _():
        pltpu.sync_copy(stbuf.at[pl.ds(0, ROWS_LAST)],
                        selfterm.at[pl.ds(row0, ROWS_LAST)])


def _mm_body(x_ref, w_ref, o_ref):
    o_ref[...] = jnp.dot(x_ref[...], w_ref[...],
                         preferred_element_type=jnp.float32)


def _relu_body(p_ref, hp_ref, u_ref):
    u_ref[...] = jnp.maximum(p_ref[0] + p_ref[1] + hp_ref[...], 0.0)


def _final_body(q_ref, g2_ref, w1_ref, o_ref):
    t = q_ref[0] + q_ref[1] + g2_ref[...]
    o_ref[...] = jnp.dot(t, w1_ref[...], preferred_element_type=jnp.float32)


def kernel(x, edge_index, W0, W1):
    f32 = jnp.float32
    edge2d = edge_index.reshape(2, NROWS2D, CHUNK)
    ones1 = jnp.ones((N_PAD,), f32)
    zeros1 = jnp.zeros((N_PAD,), f32)
    zeros16 = jnp.zeros((N_NODES, D_HID), f32)

    h0 = pl.pallas_call(
        _mm_body,
        grid=(10,),
        in_specs=[
            pl.BlockSpec((N_NODES // 10, 128), lambda i: (i, 0)),
            pl.BlockSpec((128, D_HID), lambda i: (0, 0)),
        ],
        out_specs=pl.BlockSpec((N_NODES // 10, D_HID), lambda i: (i, 0)),
        out_shape=jax.ShapeDtypeStruct((N_NODES, D_HID), f32),
    )(x, W0)

    deg = _deg_kernel(edge2d, ones1, zeros1)

    p, hp1 = _agg_kernel(h0, deg, edge2d, zeros16)

    u = pl.pallas_call(
        _relu_body,
        out_shape=jax.ShapeDtypeStruct((N_NODES, D_HID), f32),
    )(p, hp1)

    q, g2 = _agg_kernel(u, deg, edge2d, zeros16)

    out = pl.pallas_call(
        _final_body,
        out_shape=jax.ShapeDtypeStruct((N_NODES, 7), f32),
    )(q, g2, W1)

    return out


# speedup vs baseline: 1.2365x; 1.1188x over previous
import functools

import jax
import jax.numpy as jnp
from jax import lax
from jax.experimental import pallas as pl
from jax.experimental.pallas import tpu as pltpu
from jax.experimental.pallas import tpu_sc as plsc

N_NODES = 10000
N_PAD = 10240
D_HID = 16
E_EDGES = 320000
CHUNK = 128
NROWS2D = E_EDGES // CHUNK
BASECH = NROWS2D // 32
NBUF = 6
ROWS_FULL = 640
ROWS_LAST = N_NODES - 15 * ROWS_FULL

_SC_MESH = plsc.VectorSubcoreMesh(core_axis_name="c", subcore_axis_name="s")
_SC_PARAMS = pltpu.CompilerParams(use_tc_tiling_on_sc=False,
                                  needs_layout_passes=False)


def _rsqrt16(d):
    i = plsc.bitcast(d, jnp.int32)
    i = jnp.full((16,), 0x5F3759DF, jnp.int32) - lax.shift_right_logical(i, 1)
    y = plsc.bitcast(i, jnp.float32)
    half = 0.5 * d
    for _ in range(3):
        y = y * (1.5 - half * y * y)
    return y


_GATHER_DNUMS = lax.GatherDimensionNumbers(
    offset_dims=(), collapsed_slice_dims=(0,), start_index_map=(0,))


def _bcast_lane(v, k):
    idx = jnp.full((16, 1), k, jnp.int32)
    return lax.gather(v, idx, _GATHER_DNUMS, slice_sizes=(1,),
                      mode=lax.GatherScatterMode.PROMISE_IN_BOUNDS)


@functools.partial(
    pl.kernel,
    out_type=jax.ShapeDtypeStruct((2, N_PAD), jnp.float32),
    mesh=_SC_MESH,
    scratch_types=[
        pltpu.VMEM((BASECH + 1, CHUNK), jnp.int32),
        pltpu.VMEM((CHUNK,), jnp.float32),
        [pltpu.SemaphoreType.DMA for _ in range(NBUF)],
        pltpu.VMEM_SHARED((N_PAD,), jnp.float32),
    ],
    compiler_params=_SC_PARAMS,
)
def _deg_kernel(edge2d, ones_init, zeros_init, out, didx, ones_v, ssems,
                dacc):
    cid = lax.axis_index("c")
    sid = lax.axis_index("s")
    w = cid * 16 + sid
    base = w * BASECH + jnp.minimum(w, 4)
    extra = w < 4
    pltpu.sync_copy(edge2d.at[1, pl.ds(base, BASECH)],
                    didx.at[pl.ds(0, BASECH)])

    @pl.when(extra)
    def _():
        pltpu.sync_copy(edge2d.at[1, pl.ds(base + BASECH, 1)],
                        didx.at[pl.ds(BASECH, 1)])

    pltpu.sync_copy(ones_init.at[pl.ds(0, CHUNK)], ones_v)
    row0 = sid * ROWS_FULL

    @pl.when(cid == 0)
    def _():
        pltpu.sync_copy(ones_init.at[pl.ds(row0, ROWS_FULL)],
                        dacc.at[pl.ds(row0, ROWS_FULL)])

    @pl.when(cid == 1)
    def _():
        pltpu.sync_copy(zeros_init.at[pl.ds(row0, ROWS_FULL)],
                        dacc.at[pl.ds(row0, ROWS_FULL)])

    plsc.subcore_barrier()

    def body(i, carry):
        jj = i * NBUF
        sds = [
            pltpu.async_copy(ones_v, dacc.at[didx.at[jj + b]], ssems[b],
                             add=True)
            for b in range(NBUF)
        ]
        for d in sds:
            d.wait()
        return carry

    lax.fori_loop(0, BASECH // NBUF, body, 0)

    @pl.when(extra)
    def _():
        pltpu.sync_copy(ones_v, dacc.at[didx.at[BASECH]], add=True)

    plsc.subcore_barrier()
    pltpu.sync_copy(dacc.at[pl.ds(row0, ROWS_FULL)],
                    out.at[cid, pl.ds(row0, ROWS_FULL)])


def _make_agg(layer2):
    if layer2:
        in_types = (
            jax.ShapeDtypeStruct((2, N_NODES, D_HID), jnp.float32),
            jax.ShapeDtypeStruct((N_NODES, D_HID), jnp.float32),
        )
    else:
        in_types = (jax.ShapeDtypeStruct((N_NODES, D_HID), jnp.float32),)
    del in_types

    @functools.partial(
        pl.kernel,
        out_type=(
            jax.ShapeDtypeStruct((2, N_NODES, D_HID), jnp.float32),
            jax.ShapeDtypeStruct((N_NODES, D_HID), jnp.float32),
        ),
        mesh=_SC_MESH,
        scratch_types=[
            pltpu.VMEM((BASECH + 1, CHUNK), jnp.int32),
            pltpu.VMEM((BASECH + 1, CHUNK), jnp.int32),
            pltpu.VMEM((ROWS_FULL,), jnp.float32),
            pltpu.VMEM((ROWS_FULL,), jnp.float32),
            pltpu.VMEM((ROWS_FULL, D_HID), jnp.float32),
            pltpu.VMEM((ROWS_FULL, D_HID), jnp.float32),
            pltpu.VMEM((ROWS_FULL, D_HID), jnp.float32),
            pltpu.VMEM((ROWS_FULL, D_HID), jnp.float32),
            pltpu.VMEM((ROWS_FULL, D_HID), jnp.float32),
            [pltpu.VMEM((CHUNK, D_HID), jnp.float32) for _ in range(NBUF)],
            [pltpu.SemaphoreType.DMA for _ in range(NBUF)],
            [pltpu.SemaphoreType.DMA for _ in range(NBUF)],
            pltpu.VMEM_SHARED((N_PAD, D_HID), jnp.float32),
            pltpu.VMEM_SHARED((N_PAD, D_HID), jnp.float32),
        ],
        compiler_params=_SC_PARAMS,
        name="agg2" if layer2 else "agg1",
    )
    def agg(*args):
        if layer2:
            (pin, sin, deg, edge2d, zeros16, parts, selfterm, sidx, didx,
             dg0, dg1, hbuf, sbuf, abuf, obuf, stbuf, bufs, gsems, ssems,
             acc, tbl) = args
        else:
            (hin, deg, edge2d, zeros16, parts, selfterm, sidx, didx,
             dg0, dg1, hbuf, sbuf, abuf, obuf, stbuf, bufs, gsems, ssems,
             acc, tbl) = args
        cid = lax.axis_index("c")
        sid = lax.axis_index("s")
        w = cid * 16 + sid
        base = w * BASECH + jnp.minimum(w, 4)
        extra = w < 4
        row0 = sid * ROWS_FULL
        last = sid == 15
        nvec = jnp.where(last, ROWS_LAST // 16, ROWS_FULL // 16)

        pltpu.sync_copy(edge2d.at[0, pl.ds(base, BASECH)],
                        sidx.at[pl.ds(0, BASECH)])
        pltpu.sync_copy(edge2d.at[1, pl.ds(base, BASECH)],
                        didx.at[pl.ds(0, BASECH)])

        @pl.when(extra)
        def _():
            pltpu.sync_copy(edge2d.at[0, pl.ds(base + BASECH, 1)],
                            sidx.at[pl.ds(BASECH, 1)])
            pltpu.sync_copy(edge2d.at[1, pl.ds(base + BASECH, 1)],
                            didx.at[pl.ds(BASECH, 1)])

        def stage(nrows):
            pltpu.sync_copy(deg.at[0, pl.ds(row0, nrows)],
                            dg0.at[pl.ds(0, nrows)])
            pltpu.sync_copy(deg.at[1, pl.ds(row0, nrows)],
                            dg1.at[pl.ds(0, nrows)])
            if layer2:
                pltpu.sync_copy(sin.at[pl.ds(row0, nrows)],
                                hbuf.at[pl.ds(0, nrows)])
                pltpu.sync_copy(pin.at[0, pl.ds(row0, nrows)],
                                abuf.at[pl.ds(0, nrows)])
                pltpu.sync_copy(pin.at[1, pl.ds(row0, nrows)],
                                obuf.at[pl.ds(0, nrows)])
            else:
                pltpu.sync_copy(hin.at[pl.ds(row0, nrows)],
                                hbuf.at[pl.ds(0, nrows)])
            pltpu.sync_copy(zeros16.at[pl.ds(row0, nrows)],
                            acc.at[pl.ds(row0, nrows)])

        pl.when(~last)(lambda: stage(ROWS_FULL))
        pl.when(last)(lambda: stage(ROWS_LAST))

        def scale_body(i, carry):
            r = i * 16
            dis = _rsqrt16(dg0[pl.ds(r, 16)] + dg1[pl.ds(r, 16)])
            for k in range(16):
                if layer2:
                    row = jnp.maximum(
                        abuf[r + k] + obuf[r + k] + hbuf[r + k], 0.0)
                else:
                    row = hbuf[r + k]
                sbuf[r + k] = row * _bcast_lane(dis, k)
            return carry

        lax.fori_loop(0, nvec, scale_body, 0)

        @pl.when(~last)
        def _():
            pltpu.sync_copy(sbuf, tbl.at[pl.ds(row0, ROWS_FULL)])

        @pl.when(last)
        def _():
            pltpu.sync_copy(sbuf.at[pl.ds(0, ROWS_LAST)],
                            tbl.at[pl.ds(row0, ROWS_LAST)])

        plsc.subcore_barrier()

        def body(i, carry):
            jj = i * NBUF
            gds = [
                pltpu.async_copy(tbl.at[sidx.at[jj + b]], bufs[b], gsems[b])
                for b in range(NBUF)
            ]
            sds = []
            for b in range(NBUF):
                gds[b].wait()
                sds.append(
                    pltpu.async_copy(bufs[b], acc.at[didx.at[jj + b]], ssems[b],
                                     add=True))
            for d in sds:
                d.wait()
            return carry

        lax.fori_loop(0, BASECH // NBUF, body, 0)

        @pl.when(extra)
        def _():
            pltpu.sync_copy(tbl.at[sidx.at[BASECH]], bufs[0])
            pltpu.sync_copy(bufs[0], acc.at[didx.at[BASECH]], add=True)

        plsc.subcore_barrier()

        @pl.when(~last)
        def _():
            pltpu.sync_copy(acc.at[pl.ds(row0, ROWS_FULL)], abuf)

        @pl.when(last)
        def _():
            pltpu.sync_copy(acc.at[pl.ds(row0, ROWS_LAST)],
                            abuf.at[pl.ds(0, ROWS_LAST)])

        def out_body(i, carry):
            r = i * 16
            dis = _rsqrt16(dg0[pl.ds(r, 16)] + dg1[pl.ds(r, 16)])
            for k in range(16):
                d16 = _bcast_lane(dis, k)
                obuf[r + k] = abuf[r + k] * d16
                stbuf[r + k] = sbuf[r + k] * d16
            return carry

        lax.fori_loop(0, nvec, out_body, 0)

        @pl.when(~last)
        def _():
            pltpu.sync_copy(obuf, parts.at[cid, pl.ds(row0, ROWS_FULL)])

        @pl.when(last)
        def _():
            pltpu.sync_copy(obuf.at[pl.ds(0, ROWS_LAST)],
                            parts.at[cid, pl.ds(row0, ROWS_LAST)])

        @pl.when((cid == 0) & ~last)
        def _():
            pltpu.sync_copy(stbuf, selfterm.at[pl.ds(row0, ROWS_FULL)])

        @pl.when((cid == 0) & last)
        def _():
            pltpu.sync_copy(stbuf.at[pl.ds(0, ROWS_LAST)],
                            selfterm.at[pl.ds(row0, ROWS_LAST)])

    return agg


_agg1_kernel = _make_agg(False)
_agg2_kernel = _make_agg(True)


def _mm_body(x_ref, w_ref, o_ref):
    o_ref[...] = jnp.dot(x_ref[...], w_ref[...],
                         preferred_element_type=jnp.float32)


def _final_body(q_ref, g2_ref, w1_ref, o_ref):
    t = q_ref[0] + q_ref[1] + g2_ref[...]
    o_ref[...] = jnp.dot(t, w1_ref[...], preferred_element_type=jnp.float32)


def kernel(x, edge_index, W0, W1):
    f32 = jnp.float32
    edge2d = edge_index.reshape(2, NROWS2D, CHUNK)
    ones1 = jnp.ones((N_PAD,), f32)
    zeros1 = jnp.zeros((N_PAD,), f32)
    zeros16 = jnp.zeros((N_NODES, D_HID), f32)

    h0 = pl.pallas_call(
        _mm_body,
        grid=(10,),
        in_specs=[
            pl.BlockSpec((N_NODES // 10, 128), lambda i: (i, 0)),
            pl.BlockSpec((128, D_HID), lambda i: (0, 0)),
        ],
        out_specs=pl.BlockSpec((N_NODES // 10, D_HID), lambda i: (i, 0)),
        out_shape=jax.ShapeDtypeStruct((N_NODES, D_HID), f32),
    )(x, W0)

    deg = _deg_kernel(edge2d, ones1, zeros1)

    p, hp1 = _agg1_kernel(h0, deg, edge2d, zeros16)

    q, g2 = _agg2_kernel(p, hp1, deg, edge2d, zeros16)

    out = pl.pallas_call(
        _final_body,
        out_shape=jax.ShapeDtypeStruct((N_NODES, 7), f32),
    )(q, g2, W1)

    return out


# speedup vs baseline: 1.4216x; 1.1497x over previous
import functools

import jax
import jax.numpy as jnp
from jax import lax
from jax.experimental import pallas as pl
from jax.experimental.pallas import tpu as pltpu
from jax.experimental.pallas import tpu_sc as plsc

N_NODES = 10000
N_PAD = 10240
D_HID = 16
E_EDGES = 320000
CHUNK = 128
NROWS2D = E_EDGES // CHUNK
BASECH = NROWS2D // 32
NBUF = 6
ROWS_FULL = 640
ROWS_LAST = N_NODES - 15 * ROWS_FULL

_SC_MESH = plsc.VectorSubcoreMesh(core_axis_name="c", subcore_axis_name="s")
_SC_PARAMS = pltpu.CompilerParams(use_tc_tiling_on_sc=False,
                                  needs_layout_passes=False)


def _rsqrt16(d):
    i = plsc.bitcast(d, jnp.int32)
    i = jnp.full((16,), 0x5F3759DF, jnp.int32) - lax.shift_right_logical(i, 1)
    y = plsc.bitcast(i, jnp.float32)
    half = 0.5 * d
    for _ in range(3):
        y = y * (1.5 - half * y * y)
    return y


_GATHER_DNUMS = lax.GatherDimensionNumbers(
    offset_dims=(), collapsed_slice_dims=(0,), start_index_map=(0,))


def _bcast_lane(v, k):
    idx = jnp.full((16, 1), k, jnp.int32)
    return lax.gather(v, idx, _GATHER_DNUMS, slice_sizes=(1,),
                      mode=lax.GatherScatterMode.PROMISE_IN_BOUNDS)


@functools.partial(
    pl.kernel,
    out_type=jax.ShapeDtypeStruct((2, N_PAD), jnp.float32),
    mesh=_SC_MESH,
    scratch_types=[
        pltpu.VMEM((BASECH + 1, CHUNK), jnp.int32),
        pltpu.VMEM((CHUNK,), jnp.float32),
        [pltpu.SemaphoreType.DMA for _ in range(NBUF)],
        pltpu.VMEM_SHARED((N_PAD,), jnp.float32),
    ],
    compiler_params=_SC_PARAMS,
)
def _deg_kernel(edge2d, ones_init, zeros_init, out, didx, ones_v, ssems,
                dacc):
    cid = lax.axis_index("c")
    sid = lax.axis_index("s")
    w = cid * 16 + sid
    base = w * BASECH + jnp.minimum(w, 4)
    extra = w < 4
    pltpu.sync_copy(edge2d.at[1, pl.ds(base, BASECH)],
                    didx.at[pl.ds(0, BASECH)])

    @pl.when(extra)
    def _():
        pltpu.sync_copy(edge2d.at[1, pl.ds(base + BASECH, 1)],
                        didx.at[pl.ds(BASECH, 1)])

    pltpu.sync_copy(ones_init.at[pl.ds(0, CHUNK)], ones_v)
    row0 = sid * ROWS_FULL

    @pl.when(cid == 0)
    def _():
        pltpu.sync_copy(ones_init.at[pl.ds(row0, ROWS_FULL)],
                        dacc.at[pl.ds(row0, ROWS_FULL)])

    @pl.when(cid == 1)
    def _():
        pltpu.sync_copy(zeros_init.at[pl.ds(row0, ROWS_FULL)],
                        dacc.at[pl.ds(row0, ROWS_FULL)])

    plsc.subcore_barrier()

    def body(i, carry):
        jj = i * NBUF
        sds = [
            pltpu.async_copy(ones_v, dacc.at[didx.at[jj + b]], ssems[b],
                             add=True)
            for b in range(NBUF)
        ]
        for d in sds:
            d.wait()
        return carry

    lax.fori_loop(0, BASECH // NBUF, body, 0)

    @pl.when(extra)
    def _():
        pltpu.sync_copy(ones_v, dacc.at[didx.at[BASECH]], add=True)

    plsc.subcore_barrier()
    pltpu.sync_copy(dacc.at[pl.ds(row0, ROWS_FULL)],
                    out.at[cid, pl.ds(row0, ROWS_FULL)])


def _make_agg(layer2):
    if layer2:
        in_types = (
            jax.ShapeDtypeStruct((2, N_NODES, D_HID), jnp.float32),
            jax.ShapeDtypeStruct((N_NODES, D_HID), jnp.float32),
        )
    else:
        in_types = (jax.ShapeDtypeStruct((N_NODES, D_HID), jnp.float32),)
    del in_types

    @functools.partial(
        pl.kernel,
        out_type=jax.ShapeDtypeStruct((2, N_NODES, D_HID), jnp.float32),
        mesh=_SC_MESH,
        scratch_types=[
            pltpu.VMEM((BASECH + 1, CHUNK), jnp.int32),
            pltpu.VMEM((BASECH + 1, CHUNK), jnp.int32),
            pltpu.VMEM((ROWS_FULL,), jnp.float32),
            pltpu.VMEM((ROWS_FULL,), jnp.float32),
            pltpu.VMEM((ROWS_FULL, D_HID), jnp.float32),
            pltpu.VMEM((ROWS_FULL, D_HID), jnp.float32),
            pltpu.VMEM((ROWS_FULL, D_HID), jnp.float32),
            pltpu.VMEM((ROWS_FULL, D_HID), jnp.float32),
            [pltpu.VMEM((CHUNK, D_HID), jnp.float32) for _ in range(NBUF)],
            [pltpu.SemaphoreType.DMA for _ in range(NBUF)],
            [pltpu.SemaphoreType.DMA for _ in range(NBUF)],
            pltpu.VMEM_SHARED((N_PAD, D_HID), jnp.float32),
            pltpu.VMEM_SHARED((N_PAD, D_HID), jnp.float32),
        ],
        compiler_params=_SC_PARAMS,
        name="agg2" if layer2 else "agg1",
    )
    def agg(*args):
        if layer2:
            (pin, deg, edge2d, zeros16, parts, sidx, didx,
             dg0, dg1, hbuf, sbuf, abuf, obuf, bufs, gsems, ssems,
             acc, tbl) = args
        else:
            (hin, deg, edge2d, zeros16, parts, sidx, didx,
             dg0, dg1, hbuf, sbuf, abuf, obuf, bufs, gsems, ssems,
             acc, tbl) = args
        cid = lax.axis_index("c")
        sid = lax.axis_index("s")
        w = cid * 16 + sid
        base = w * BASECH + jnp.minimum(w, 4)
        extra = w < 4
        row0 = sid * ROWS_FULL
        last = sid == 15
        nvec = jnp.where(last, ROWS_LAST // 16, ROWS_FULL // 16)

        descs = [
            pltpu.async_copy(edge2d.at[0, pl.ds(base, BASECH)],
                             sidx.at[pl.ds(0, BASECH)], gsems[0]),
            pltpu.async_copy(edge2d.at[1, pl.ds(base, BASECH)],
                             didx.at[pl.ds(0, BASECH)], gsems[1]),
        ]

        @pl.when(extra)
        def _():
            pltpu.sync_copy(edge2d.at[0, pl.ds(base + BASECH, 1)],
                            sidx.at[pl.ds(BASECH, 1)])
            pltpu.sync_copy(edge2d.at[1, pl.ds(base + BASECH, 1)],
                            didx.at[pl.ds(BASECH, 1)])

        def stage(nrows):
            ds_ = [
                pltpu.async_copy(deg.at[0, pl.ds(row0, nrows)],
                                 dg0.at[pl.ds(0, nrows)], gsems[2]),
                pltpu.async_copy(deg.at[1, pl.ds(row0, nrows)],
                                 dg1.at[pl.ds(0, nrows)], gsems[3]),
                pltpu.async_copy(zeros16.at[pl.ds(row0, nrows)],
                                 acc.at[pl.ds(row0, nrows)], gsems[4]),
            ]
            if layer2:
                ds_.append(
                    pltpu.async_copy(pin.at[0, pl.ds(row0, nrows)],
                                     abuf.at[pl.ds(0, nrows)], ssems[0]))
                ds_.append(
                    pltpu.async_copy(pin.at[1, pl.ds(row0, nrows)],
                                     hbuf.at[pl.ds(0, nrows)], ssems[1]))
            else:
                ds_.append(
                    pltpu.async_copy(hin.at[pl.ds(row0, nrows)],
                                     hbuf.at[pl.ds(0, nrows)], ssems[0]))
            for d in ds_:
                d.wait()

        pl.when(~last)(lambda: stage(ROWS_FULL))
        pl.when(last)(lambda: stage(ROWS_LAST))
        for d in descs:
            d.wait()

        def scale_body(i, carry):
            r = i * 16
            dis = _rsqrt16(dg0[pl.ds(r, 16)] + dg1[pl.ds(r, 16)])
            for k in range(16):
                if layer2:
                    row = jnp.maximum(abuf[r + k] + hbuf[r + k], 0.0)
                else:
                    row = hbuf[r + k]
                sbuf[r + k] = row * _bcast_lane(dis, k)
            return carry

        lax.fori_loop(0, nvec, scale_body, 0)

        @pl.when(~last)
        def _():
            pltpu.sync_copy(sbuf, tbl.at[pl.ds(row0, ROWS_FULL)])

        @pl.when(last)
        def _():
            pltpu.sync_copy(sbuf.at[pl.ds(0, ROWS_LAST)],
                            tbl.at[pl.ds(row0, ROWS_LAST)])

        plsc.subcore_barrier()

        def body(i, carry):
            jj = i * NBUF
            gds = [
                pltpu.async_copy(tbl.at[sidx.at[jj + b]], bufs[b], gsems[b])
                for b in range(NBUF)
            ]
            sds = []
            for b in range(NBUF):
                gds[b].wait()
                sds.append(
                    pltpu.async_copy(bufs[b], acc.at[didx.at[jj + b]], ssems[b],
                                     add=True))
            for d in sds:
                d.wait()
            return carry

        lax.fori_loop(0, BASECH // NBUF, body, 0)

        @pl.when(extra)
        def _():
            pltpu.sync_copy(tbl.at[sidx.at[BASECH]], bufs[0])
            pltpu.sync_copy(bufs[0], acc.at[didx.at[BASECH]], add=True)

        plsc.subcore_barrier()

        @pl.when(~last)
        def _():
            pltpu.sync_copy(acc.at[pl.ds(row0, ROWS_FULL)], abuf)

        @pl.when(last)
        def _():
            pltpu.sync_copy(acc.at[pl.ds(row0, ROWS_LAST)],
                            abuf.at[pl.ds(0, ROWS_LAST)])

        def out_body0(i, carry):
            r = i * 16
            dis = _rsqrt16(dg0[pl.ds(r, 16)] + dg1[pl.ds(r, 16)])
            for k in range(16):
                obuf[r + k] = (abuf[r + k] + sbuf[r + k]) * _bcast_lane(
                    dis, k)
            return carry

        def out_body1(i, carry):
            r = i * 16
            dis = _rsqrt16(dg0[pl.ds(r, 16)] + dg1[pl.ds(r, 16)])
            for k in range(16):
                obuf[r + k] = abuf[r + k] * _bcast_lane(dis, k)
            return carry

        @pl.when(cid == 0)
        def _():
            lax.fori_loop(0, nvec, out_body0, 0)

        @pl.when(cid == 1)
        def _():
            lax.fori_loop(0, nvec, out_body1, 0)

        @pl.when(~last)
        def _():
            pltpu.sync_copy(obuf, parts.at[cid, pl.ds(row0, ROWS_FULL)])

        @pl.when(last)
        def _():
            pltpu.sync_copy(obuf.at[pl.ds(0, ROWS_LAST)],
                            parts.at[cid, pl.ds(row0, ROWS_LAST)])

    return agg


_agg1_kernel = _make_agg(False)
_agg2_kernel = _make_agg(True)


def _mm_body(x_ref, w_ref, o_ref):
    o_ref[...] = jnp.dot(x_ref[...], w_ref[...],
                         preferred_element_type=jnp.float32)


def _final_body(q_ref, w1_ref, o_ref):
    t = q_ref[0] + q_ref[1]
    o_ref[...] = jnp.dot(t, w1_ref[...], preferred_element_type=jnp.float32)


def kernel(x, edge_index, W0, W1):
    f32 = jnp.float32
    edge2d = edge_index.reshape(2, NROWS2D, CHUNK)
    ones1 = jnp.ones((N_PAD,), f32)
    zeros1 = jnp.zeros((N_PAD,), f32)
    zeros16 = jnp.zeros((N_NODES, D_HID), f32)

    h0 = pl.pallas_call(
        _mm_body,
        grid=(10,),
        in_specs=[
            pl.BlockSpec((N_NODES // 10, 128), lambda i: (i, 0)),
            pl.BlockSpec((128, D_HID), lambda i: (0, 0)),
        ],
        out_specs=pl.BlockSpec((N_NODES // 10, D_HID), lambda i: (i, 0)),
        out_shape=jax.ShapeDtypeStruct((N_NODES, D_HID), f32),
    )(x, W0)

    deg = _deg_kernel(edge2d, ones1, zeros1)

    p = _agg1_kernel(h0, deg, edge2d, zeros16)

    q = _agg2_kernel(p, deg, edge2d, zeros16)

    out = pl.pallas_call(
        _final_body,
        out_shape=jax.ShapeDtypeStruct((N_NODES, 7), f32),
    )(q, W1)

    return out


# speedup vs baseline: 1.4216x; 1.0000x over previous
import functools

import jax
import jax.numpy as jnp
from jax import lax
from jax.experimental import pallas as pl
from jax.experimental.pallas import tpu as pltpu
from jax.experimental.pallas import tpu_sc as plsc

N_NODES = 10000
N_PAD = 10240
D_HID = 16
E_EDGES = 320000
CHUNK = 128
NROWS2D = E_EDGES // CHUNK
BASECH = NROWS2D // 32
NBUF = 6
ROWS_FULL = 640
ROWS_LAST = N_NODES - 15 * ROWS_FULL

_SC_MESH = plsc.VectorSubcoreMesh(core_axis_name="c", subcore_axis_name="s")
_SC_PARAMS = pltpu.CompilerParams(use_tc_tiling_on_sc=False,
                                  needs_layout_passes=False)


def _rsqrt16(d):
    i = plsc.bitcast(d, jnp.int32)
    i = jnp.full((16,), 0x5F3759DF, jnp.int32) - lax.shift_right_logical(i, 1)
    y = plsc.bitcast(i, jnp.float32)
    half = 0.5 * d
    for _ in range(3):
        y = y * (1.5 - half * y * y)
    return y


_GATHER_DNUMS = lax.GatherDimensionNumbers(
    offset_dims=(), collapsed_slice_dims=(0,), start_index_map=(0,))


def _bcast_lane(v, k):
    idx = jnp.full((16, 1), k, jnp.int32)
    return lax.gather(v, idx, _GATHER_DNUMS, slice_sizes=(1,),
                      mode=lax.GatherScatterMode.PROMISE_IN_BOUNDS)


@functools.partial(
    pl.kernel,
    out_type=jax.ShapeDtypeStruct((2, N_PAD), jnp.float32),
    mesh=_SC_MESH,
    scratch_types=[
        pltpu.VMEM((BASECH + 1, CHUNK), jnp.int32),
        pltpu.VMEM((CHUNK,), jnp.float32),
        [pltpu.SemaphoreType.DMA for _ in range(NBUF)],
        pltpu.VMEM_SHARED((N_PAD,), jnp.float32),
    ],
    compiler_params=_SC_PARAMS,
)
def _deg_kernel(edge2d, ones_init, zeros_init, out, didx, ones_v, ssems,
                dacc):
    cid = lax.axis_index("c")
    sid = lax.axis_index("s")
    w = cid * 16 + sid
    base = w * BASECH + jnp.minimum(w, 4)
    extra = w < 4
    pltpu.sync_copy(edge2d.at[1, pl.ds(base, BASECH)],
                    didx.at[pl.ds(0, BASECH)])

    @pl.when(extra)
    def _():
        pltpu.sync_copy(edge2d.at[1, pl.ds(base + BASECH, 1)],
                        didx.at[pl.ds(BASECH, 1)])

    pltpu.sync_copy(ones_init.at[pl.ds(0, CHUNK)], ones_v)
    row0 = sid * ROWS_FULL

    @pl.when(cid == 0)
    def _():
        pltpu.sync_copy(ones_init.at[pl.ds(row0, ROWS_FULL)],
                        dacc.at[pl.ds(row0, ROWS_FULL)])

    @pl.when(cid == 1)
    def _():
        pltpu.sync_copy(zeros_init.at[pl.ds(row0, ROWS_FULL)],
                        dacc.at[pl.ds(row0, ROWS_FULL)])

    plsc.subcore_barrier()

    def body(i, carry):
        jj = i * NBUF
        sds = [
            pltpu.async_copy(ones_v, dacc.at[didx.at[jj + b]], ssems[b],
                             add=True)
            for b in range(NBUF)
        ]
        for d in sds:
            d.wait()
        return carry

    lax.fori_loop(0, BASECH // NBUF, body, 0)

    @pl.when(extra)
    def _():
        pltpu.sync_copy(ones_v, dacc.at[didx.at[BASECH]], add=True)

    plsc.subcore_barrier()
    pltpu.sync_copy(dacc.at[pl.ds(row0, ROWS_FULL)],
                    out.at[cid, pl.ds(row0, ROWS_FULL)])


def _make_agg(layer2):
    if layer2:
        in_types = (
            jax.ShapeDtypeStruct((2, N_NODES, D_HID), jnp.float32),
            jax.ShapeDtypeStruct((N_NODES, D_HID), jnp.float32),
        )
    else:
        in_types = (jax.ShapeDtypeStruct((N_NODES, D_HID), jnp.float32),)
    del in_types

    @functools.partial(
        pl.kernel,
        out_type=jax.ShapeDtypeStruct((2, N_NODES, D_HID), jnp.float32),
        mesh=_SC_MESH,
        scratch_types=[
            pltpu.VMEM((BASECH + 1, CHUNK), jnp.int32),
            pltpu.VMEM((BASECH + 1, CHUNK), jnp.int32),
            pltpu.VMEM((ROWS_FULL,), jnp.float32),
            pltpu.VMEM((ROWS_FULL,), jnp.float32),
            pltpu.VMEM((ROWS_FULL, D_HID), jnp.float32),
            pltpu.VMEM((ROWS_FULL, D_HID), jnp.float32),
            pltpu.VMEM((ROWS_FULL, D_HID), jnp.float32),
            pltpu.VMEM((ROWS_FULL, D_HID), jnp.float32),
            [pltpu.VMEM((CHUNK, D_HID), jnp.float32) for _ in range(NBUF)],
            [pltpu.SemaphoreType.DMA for _ in range(NBUF)],
            [pltpu.SemaphoreType.DMA for _ in range(NBUF)],
            pltpu.VMEM_SHARED((N_PAD, D_HID), jnp.float32),
            pltpu.VMEM_SHARED((N_PAD, D_HID), jnp.float32),
        ],
        compiler_params=_SC_PARAMS,
        name="agg2" if layer2 else "agg1",
    )
    def agg(*args):
        if layer2:
            (pin, deg, edge2d, zeros16, parts, sidx, didx,
             dg0, dg1, hbuf, sbuf, abuf, obuf, bufs, gsems, ssems,
             acc, tbl) = args
        else:
            (hin, deg, edge2d, zeros16, parts, sidx, didx,
             dg0, dg1, hbuf, sbuf, abuf, obuf, bufs, gsems, ssems,
             acc, tbl) = args
        cid = lax.axis_index("c")
        sid = lax.axis_index("s")
        w = cid * 16 + sid
        base = w * BASECH + jnp.minimum(w, 4)
        extra = w < 4
        row0 = sid * ROWS_FULL
        last = sid == 15
        nvec = jnp.where(last, ROWS_LAST // 16, ROWS_FULL // 16)

        descs = [
            pltpu.async_copy(edge2d.at[0, pl.ds(base, BASECH)],
                             sidx.at[pl.ds(0, BASECH)], gsems[0]),
            pltpu.async_copy(edge2d.at[1, pl.ds(base, BASECH)],
                             didx.at[pl.ds(0, BASECH)], gsems[1]),
        ]

        @pl.when(extra)
        def _():
            pltpu.sync_copy(edge2d.at[0, pl.ds(base + BASECH, 1)],
                            sidx.at[pl.ds(BASECH, 1)])
            pltpu.sync_copy(edge2d.at[1, pl.ds(base + BASECH, 1)],
                            didx.at[pl.ds(BASECH, 1)])

        def stage(nrows):
            ds_ = [
                pltpu.async_copy(deg.at[0, pl.ds(row0, nrows)],
                                 dg0.at[pl.ds(0, nrows)], gsems[2]),
                pltpu.async_copy(deg.at[1, pl.ds(row0, nrows)],
                                 dg1.at[pl.ds(0, nrows)], gsems[3]),
                pltpu.async_copy(zeros16.at[pl.ds(row0, nrows)],
                                 acc.at[pl.ds(row0, nrows)], gsems[4]),
            ]
            if layer2:
                ds_.append(
                    pltpu.async_copy(pin.at[0, pl.ds(row0, nrows)],
                                     abuf.at[pl.ds(0, nrows)], ssems[0]))
                ds_.append(
                    pltpu.async_copy(pin.at[1, pl.ds(row0, nrows)],
                                     hbuf.at[pl.ds(0, nrows)], ssems[1]))
            else:
                ds_.append(
                    pltpu.async_copy(hin.at[pl.ds(row0, nrows)],
                                     hbuf.at[pl.ds(0, nrows)], ssems[0]))
            for d in ds_:
                d.wait()

        pl.when(~last)(lambda: stage(ROWS_FULL))
        pl.when(last)(lambda: stage(ROWS_LAST))
        for d in descs:
            d.wait()

        def scale_body(i, carry):
            r = i * 16
            dis = _rsqrt16(dg0[pl.ds(r, 16)] + dg1[pl.ds(r, 16)])
            for k in range(16):
                if layer2:
                    row = jnp.maximum(abuf[r + k] + hbuf[r + k], 0.0)
                else:
                    row = hbuf[r + k]
                sbuf[r + k] = row * _bcast_lane(dis, k)
            return carry

        lax.fori_loop(0, nvec, scale_body, 0)

        @pl.when(~last)
        def _():
            pltpu.sync_copy(sbuf, tbl.at[pl.ds(row0, ROWS_FULL)])

        @pl.when(last)
        def _():
            pltpu.sync_copy(sbuf.at[pl.ds(0, ROWS_LAST)],
                            tbl.at[pl.ds(row0, ROWS_LAST)])

        plsc.subcore_barrier()

        def drain_scatter(b):
            pltpu.make_async_copy(zeros16.at[pl.ds(0, CHUNK)], bufs[b],
                                  ssems[b]).wait()

        def body(i, carry):
            jj = i * NBUF

            @pl.when(i > 0)
            def _():
                for b in range(NBUF):
                    drain_scatter(b)

            gds = [
                pltpu.async_copy(tbl.at[sidx.at[jj + b]], bufs[b], gsems[b])
                for b in range(NBUF)
            ]
            for b in range(NBUF):
                gds[b].wait()
                pltpu.async_copy(bufs[b], acc.at[didx.at[jj + b]], ssems[b],
                                 add=True)
            return carry

        lax.fori_loop(0, BASECH // NBUF, body, 0)
        for b in range(NBUF):
            drain_scatter(b)

        @pl.when(extra)
        def _():
            pltpu.sync_copy(tbl.at[sidx.at[BASECH]], bufs[0])
            pltpu.sync_copy(bufs[0], acc.at[didx.at[BASECH]], add=True)

        plsc.subcore_barrier()

        @pl.when(~last)
        def _():
            pltpu.sync_copy(acc.at[pl.ds(row0, ROWS_FULL)], abuf)

        @pl.when(last)
        def _():
            pltpu.sync_copy(acc.at[pl.ds(row0, ROWS_LAST)],
                            abuf.at[pl.ds(0, ROWS_LAST)])

        def out_body0(i, carry):
            r = i * 16
            dis = _rsqrt16(dg0[pl.ds(r, 16)] + dg1[pl.ds(r, 16)])
            for k in range(16):
                obuf[r + k] = (abuf[r + k] + sbuf[r + k]) * _bcast_lane(
                    dis, k)
            return carry

        def out_body1(i, carry):
            r = i * 16
            dis = _rsqrt16(dg0[pl.ds(r, 16)] + dg1[pl.ds(r, 16)])
            for k in range(16):
                obuf[r + k] = abuf[r + k] * _bcast_lane(dis, k)
            return carry

        @pl.when(cid == 0)
        def _():
            lax.fori_loop(0, nvec, out_body0, 0)

        @pl.when(cid == 1)
        def _():
            lax.fori_loop(0, nvec, out_body1, 0)

        @pl.when(~last)
        def _():
            pltpu.sync_copy(obuf, parts.at[cid, pl.ds(row0, ROWS_FULL)])

        @pl.when(last)
        def _():
            pltpu.sync_copy(obuf.at[pl.ds(0, ROWS_LAST)],
                            parts.at[cid, pl.ds(row0, ROWS_LAST)])

    return agg


_agg1_kernel = _make_agg(False)
_agg2_kernel = _make_agg(True)


def _mm_body(x_ref, w_ref, o_ref):
    o_ref[...] = jnp.dot(x_ref[...], w_ref[...],
                         preferred_element_type=jnp.float32)


def _final_body(q_ref, w1_ref, o_ref):
    t = q_ref[0] + q_ref[1]
    o_ref[...] = jnp.dot(t, w1_ref[...], preferred_element_type=jnp.float32)


def kernel(x, edge_index, W0, W1):
    f32 = jnp.float32
    edge2d = edge_index.reshape(2, NROWS2D, CHUNK)
    ones1 = jnp.ones((N_PAD,), f32)
    zeros1 = jnp.zeros((N_PAD,), f32)
    zeros16 = jnp.zeros((N_NODES, D_HID), f32)

    h0 = pl.pallas_call(
        _mm_body,
        grid=(10,),
        in_specs=[
            pl.BlockSpec((N_NODES // 10, 128), lambda i: (i, 0)),
            pl.BlockSpec((128, D_HID), lambda i: (0, 0)),
        ],
        out_specs=pl.BlockSpec((N_NODES // 10, D_HID), lambda i: (i, 0)),
        out_shape=jax.ShapeDtypeStruct((N_NODES, D_HID), f32),
    )(x, W0)

    deg = _deg_kernel(edge2d, ones1, zeros1)

    p = _agg1_kernel(h0, deg, edge2d, zeros16)

    q = _agg2_kernel(p, deg, edge2d, zeros16)

    out = pl.pallas_call(
        _final_body,
        out_shape=jax.ShapeDtypeStruct((N_NODES, 7), f32),
    )(q, W1)

    return out
